# baseline (speedup 1.0000x reference)
import functools

import jax
import jax.numpy as jnp
from jax import lax
from jax.experimental import pallas as pl
from jax.experimental.pallas import tpu as pltpu
from jax.experimental.pallas import tpu_sc as plsc

_NC = 2
_NS = 16
_NW = _NC * _NS
_LANES = 16


@functools.lru_cache(maxsize=None)
def _build_sc_pool(B, L, D, HL):
    H = L // HL
    BPW = B // _NW
    RPW = BPW * H
    KD = D // _LANES

    mesh = plsc.VectorSubcoreMesh(
        core_axis_name="c", subcore_axis_name="s",
        num_cores=_NC, num_subcores=_NS)

    def body(idx_hbm, table_hbm, out_hbm, idx_v, rows_v, acc_v, sem0, sem1):
        wid = lax.axis_index("s") * _NC + lax.axis_index("c")
        row0 = wid * RPW
        sems = (sem0, sem1)

        pltpu.sync_copy(idx_hbm.at[pl.ds(row0, RPW)], idx_v)

        def gcopy(h, p):
            return pltpu.make_async_copy(
                table_hbm.at[idx_v.at[h]], rows_v.at[p], sems[p])

        gcopy(0, 0).start()
        gcopy(1, 1).start()

        def outer(i, carry):
            acc = (jnp.zeros((_LANES,), jnp.float32),) * KD
            for b in range(H):
                h = i * H + b
                gcopy(h, b).wait()

                def inner(j, a):
                    return tuple(
                        a[k] + rows_v[b, j, pl.ds(k * _LANES, _LANES)]
                        for k in range(KD))
                acc = lax.fori_loop(0, HL, inner, acc, unroll=4)

                @pl.when(h + H < RPW)
                def _():
                    gcopy(h + H, b).start()
            for k in range(KD):
                acc_v[i, pl.ds(k * _LANES, _LANES)] = acc[k]
            return carry

        lax.fori_loop(0, BPW, outer, 0)
        pltpu.sync_copy(acc_v, out_hbm.at[pl.ds(wid * BPW, BPW)])

    return pl.kernel(
        body,
        out_type=jax.ShapeDtypeStruct((B, D), jnp.float32),
        mesh=mesh,
        compiler_params=pltpu.CompilerParams(use_tc_tiling_on_sc=False),
        scratch_types=[
            pltpu.VMEM((RPW, HL), jnp.int32),
            pltpu.VMEM((H, HL, D), jnp.float32),
            pltpu.VMEM((BPW, D), jnp.float32),
            pltpu.SemaphoreType.DMA,
            pltpu.SemaphoreType.DMA,
        ],
    )


def _tc_head(sums, lens, w, b):
    def body(s_ref, l_ref, w_ref, b_ref, o_ref):
        pooled = s_ref[...] / l_ref[...]
        o_ref[...] = jnp.dot(
            pooled, w_ref[...], preferred_element_type=jnp.float32) + b_ref[...]

    return pl.pallas_call(
        body,
        out_shape=jax.ShapeDtypeStruct((sums.shape[0], w.shape[1]), jnp.float32),
    )(sums, lens, w, b)


def kernel(train_x, train_x_len, emb_table, W4, b4):
    B, L = train_x.shape
    D = emb_table.shape[1]
    C = W4.shape[0]
    HL = 100
    sc_pool = _build_sc_pool(B, L, D, HL)
    idx = train_x.reshape(B * (L // HL), HL).astype(jnp.int32)
    sums = sc_pool(idx, emb_table)
    lens = train_x_len.reshape(B, 1).astype(jnp.float32)
    return _tc_head(sums, lens, W4.T, b4.reshape(1, C))
